# Initial kernel scaffold; baseline (speedup 1.0000x reference)
import jax
import jax.numpy as jnp
from jax.experimental import pallas as pl


def kernel(state_embedding, typed_edges, W):
    raise NotImplementedError("write your pallas kernel here")


# speedup vs baseline: 5.4635x; 5.4635x over previous
import functools

import jax
import jax.numpy as jnp
from jax import lax
from jax.experimental import pallas as pl
from jax.experimental.pallas import tpu as pltpu
from jax.experimental.pallas import tpu_sc as plsc

N = 10000
E = 320000
H = 128
T = 4

NC = 2
NS = 16
NW = NC * NS

BATCH = 128
PAIRS = 2 * E
NB = 160
IDX_CHUNK = 16
PAIRS_PAD = NW * NB * BATCH
ROWS_PER_TILE = 640
ACC_ROWS = NS * ROWS_PER_TILE
DUMP = N


def _yy_body(emb_ref, w_ref, out_ref):
    out_ref[...] = lax.dot_general(
        emb_ref[...], w_ref[0],
        (((1,), (1,)), ((), ())),
        preferred_element_type=jnp.float32,
    )


def _yy_matmul(emb, wstk):
    blk = 400
    return pl.pallas_call(
        _yy_body,
        grid=(8, N // blk),
        in_specs=[
            pl.BlockSpec((blk, H), lambda k, n: (n, 0)),
            pl.BlockSpec((1, H, H), lambda k, n: (k, 0, 0)),
        ],
        out_specs=pl.BlockSpec((blk, H), lambda k, n: (k * (N // blk) + n, 0)),
        out_shape=jax.ShapeDtypeStruct((8 * N, H), jnp.float32),
    )(emb, wstk)


def _combine_body(p_ref, o_ref):
    o_ref[...] = p_ref[0] + p_ref[1]


def _combine(partials):
    blk = 400
    return pl.pallas_call(
        _combine_body,
        grid=(N // blk,),
        in_specs=[pl.BlockSpec((2, blk, H), lambda i: (0, i, 0))],
        out_specs=pl.BlockSpec((blk, H), lambda i: (i, 0)),
        out_shape=jax.ShapeDtypeStruct((N, H), jnp.float32),
    )(partials)


def _sc_scatter(yy, g3, d3):
    mesh = plsc.VectorSubcoreMesh(core_axis_name="c", subcore_axis_name="s")

    @functools.partial(
        pl.kernel,
        out_type=jax.ShapeDtypeStruct((NC, ACC_ROWS, H), jnp.float32),
        mesh=mesh,
        scratch_types=[
            pltpu.VMEM((IDX_CHUNK, BATCH), jnp.int32),
            pltpu.VMEM((IDX_CHUNK, BATCH), jnp.int32),
            pltpu.VMEM((BATCH, H), jnp.float32),
            pltpu.VMEM_SHARED((ACC_ROWS, H), jnp.float32),
            pltpu.SemaphoreType.DMA,
        ],
    )
    def k(yy_hbm, g_hbm, d_hbm, out_hbm, g_v, d_v, rows_v, acc, sem):
        cid = lax.axis_index("c")
        sid = lax.axis_index("s")
        wid = cid * NS + sid

        zero = jnp.zeros((16,), jnp.float32)

        @pl.loop(0, BATCH)
        def _(r):
            @pl.loop(0, H // 16)
            def _(c):
                rows_v[r, pl.ds(c * 16, 16)] = zero

        @pl.loop(0, ROWS_PER_TILE // BATCH)
        def _(b):
            pltpu.sync_copy(
                rows_v, acc.at[pl.ds(sid * ROWS_PER_TILE + b * BATCH, BATCH)]
            )

        plsc.subcore_barrier()

        @pl.loop(0, NB // IDX_CHUNK)
        def _(c):
            pltpu.sync_copy(g_hbm.at[wid, pl.ds(c * IDX_CHUNK, IDX_CHUNK)], g_v)
            pltpu.sync_copy(d_hbm.at[wid, pl.ds(c * IDX_CHUNK, IDX_CHUNK)], d_v)

            @pl.loop(0, IDX_CHUNK)
            def _(j):
                pltpu.async_copy(yy_hbm.at[g_v.at[j]], rows_v, sem).wait()
                pltpu.sync_copy(rows_v, acc.at[d_v.at[j]], add=True)

        plsc.subcore_barrier()

        @pl.loop(0, ROWS_PER_TILE // BATCH)
        def _(b):
            base = sid * ROWS_PER_TILE + b * BATCH
            pltpu.sync_copy(
                acc.at[pl.ds(base, BATCH)], out_hbm.at[cid, pl.ds(base, BATCH)]
            )

    return k(yy, g3, d3)


@jax.jit
def kernel(state_embedding, typed_edges, W):
    et = typed_edges[0] - 1
    src = typed_edges[1]
    dst = typed_edges[2]

    g = jnp.concatenate([et * N + src, 4 * N + et * N + dst])
    d = jnp.concatenate([dst, dst])
    pad = PAIRS_PAD - PAIRS
    g = jnp.concatenate([g, jnp.zeros((pad,), jnp.int32)])
    d = jnp.concatenate([d, jnp.full((pad,), DUMP, jnp.int32)])
    g3 = g.reshape(NW, NB, BATCH)
    d3 = d.reshape(NW, NB, BATCH)

    wstk = jnp.concatenate([W[:, :, :H], W[:, :, H:]], axis=0)

    yy = _yy_matmul(state_embedding, wstk)
    partials = _sc_scatter(yy, g3, d3)
    return _combine(partials)


# speedup vs baseline: 5.9769x; 1.0940x over previous
import functools

import jax
import jax.numpy as jnp
from jax import lax
from jax.experimental import pallas as pl
from jax.experimental.pallas import tpu as pltpu
from jax.experimental.pallas import tpu_sc as plsc

N = 10000
E = 320000
H = 128
T = 4

NC = 2
NS = 16
NW = NC * NS

BATCH = 128
PAIRS = 2 * E
NB = 160
IDX_CHUNK = 16
PAIRS_PAD = NW * NB * BATCH
ROWS_PER_TILE = 640
ACC_ROWS = NS * ROWS_PER_TILE
DUMP = N


def _yy_body(emb_ref, w_ref, out_ref):
    out_ref[...] = lax.dot_general(
        emb_ref[...], w_ref[0],
        (((1,), (1,)), ((), ())),
        preferred_element_type=jnp.float32,
    )


def _yy_matmul(emb, wstk):
    blk = 400
    return pl.pallas_call(
        _yy_body,
        grid=(8, N // blk),
        in_specs=[
            pl.BlockSpec((blk, H), lambda k, n: (n, 0)),
            pl.BlockSpec((1, H, H), lambda k, n: (k, 0, 0)),
        ],
        out_specs=pl.BlockSpec((blk, H), lambda k, n: (k * (N // blk) + n, 0)),
        out_shape=jax.ShapeDtypeStruct((8 * N, H), jnp.float32),
    )(emb, wstk)


def _combine_body(p_ref, o_ref):
    o_ref[...] = p_ref[0] + p_ref[1]


def _combine(partials):
    blk = 400
    return pl.pallas_call(
        _combine_body,
        grid=(N // blk,),
        in_specs=[pl.BlockSpec((2, blk, H), lambda i: (0, i, 0))],
        out_specs=pl.BlockSpec((blk, H), lambda i: (i, 0)),
        out_shape=jax.ShapeDtypeStruct((N, H), jnp.float32),
    )(partials)


def _sc_scatter(yy, g3, d3):
    mesh = plsc.VectorSubcoreMesh(core_axis_name="c", subcore_axis_name="s")

    @functools.partial(
        pl.kernel,
        out_type=jax.ShapeDtypeStruct((NC, ACC_ROWS, H), jnp.float32),
        mesh=mesh,
        scratch_types=[
            pltpu.VMEM((IDX_CHUNK, BATCH), jnp.int32),
            pltpu.VMEM((IDX_CHUNK, BATCH), jnp.int32),
            pltpu.VMEM((BATCH, H), jnp.float32),
            pltpu.VMEM((BATCH, H), jnp.float32),
            pltpu.VMEM_SHARED((ACC_ROWS, H), jnp.float32),
            pltpu.SemaphoreType.DMA,
            pltpu.SemaphoreType.DMA,
        ],
    )
    def k(yy_hbm, g_hbm, d_hbm, out_hbm, g_v, d_v, rows_v, rows_w, acc, sem0, sem1):
        cid = lax.axis_index("c")
        sid = lax.axis_index("s")
        wid = cid * NS + sid

        zero = jnp.zeros((16,), jnp.float32)

        @pl.loop(0, BATCH)
        def _(r):
            @pl.loop(0, H // 16)
            def _(c):
                rows_v[r, pl.ds(c * 16, 16)] = zero

        @pl.loop(0, ROWS_PER_TILE // BATCH)
        def _(b):
            pltpu.sync_copy(
                rows_v, acc.at[pl.ds(sid * ROWS_PER_TILE + b * BATCH, BATCH)]
            )

        plsc.subcore_barrier()

        def gather(j, buf, sem):
            return pltpu.make_async_copy(yy_hbm.at[g_v.at[j]], buf, sem)

        @pl.loop(0, NB // IDX_CHUNK)
        def _(c):
            pltpu.sync_copy(g_hbm.at[wid, pl.ds(c * IDX_CHUNK, IDX_CHUNK)], g_v)
            pltpu.sync_copy(d_hbm.at[wid, pl.ds(c * IDX_CHUNK, IDX_CHUNK)], d_v)

            gather(0, rows_v, sem0).start()
            gather(1, rows_w, sem1).start()

            @pl.loop(0, IDX_CHUNK // 2 - 1)
            def _(b2):
                j0 = 2 * b2
                gather(j0, rows_v, sem0).wait()
                pltpu.sync_copy(rows_v, acc.at[d_v.at[j0]], add=True)
                gather(j0 + 2, rows_v, sem0).start()
                gather(j0 + 1, rows_w, sem1).wait()
                pltpu.sync_copy(rows_w, acc.at[d_v.at[j0 + 1]], add=True)
                gather(j0 + 3, rows_w, sem1).start()

            gather(IDX_CHUNK - 2, rows_v, sem0).wait()
            pltpu.sync_copy(rows_v, acc.at[d_v.at[IDX_CHUNK - 2]], add=True)
            gather(IDX_CHUNK - 1, rows_w, sem1).wait()
            pltpu.sync_copy(rows_w, acc.at[d_v.at[IDX_CHUNK - 1]], add=True)

        plsc.subcore_barrier()

        @pl.loop(0, ROWS_PER_TILE // BATCH)
        def _(b):
            base = sid * ROWS_PER_TILE + b * BATCH
            pltpu.sync_copy(
                acc.at[pl.ds(base, BATCH)], out_hbm.at[cid, pl.ds(base, BATCH)]
            )

    return k(yy, g3, d3)


@jax.jit
def kernel(state_embedding, typed_edges, W):
    et = typed_edges[0] - 1
    src = typed_edges[1]
    dst = typed_edges[2]

    g = jnp.concatenate([et * N + src, 4 * N + et * N + dst])
    d = jnp.concatenate([dst, dst])
    pad = PAIRS_PAD - PAIRS
    g = jnp.concatenate([g, jnp.zeros((pad,), jnp.int32)])
    d = jnp.concatenate([d, jnp.full((pad,), DUMP, jnp.int32)])
    g3 = g.reshape(NW, NB, BATCH)
    d3 = d.reshape(NW, NB, BATCH)

    wstk = jnp.concatenate([W[:, :, :H], W[:, :, H:]], axis=0)

    yy = _yy_matmul(state_embedding, wstk)
    partials = _sc_scatter(yy, g3, d3)
    return _combine(partials)


# speedup vs baseline: 6.8924x; 1.1532x over previous
import functools

import jax
import jax.numpy as jnp
from jax import lax
from jax.experimental import pallas as pl
from jax.experimental.pallas import tpu as pltpu
from jax.experimental.pallas import tpu_sc as plsc

N = 10000
E = 320000
H = 128
T = 4

NC = 2
NS = 16
NW = NC * NS

BATCH = 128
PAIRS = 2 * E
NB = 160
IDX_CHUNK = 16
PAIRS_PAD = NW * NB * BATCH
ROWS_PER_TILE = 640
ACC_ROWS = NS * ROWS_PER_TILE
DUMP = N


def _yy_body(emb_ref, w_ref, out_ref):
    out_ref[...] = lax.dot_general(
        emb_ref[...], w_ref[0],
        (((1,), (1,)), ((), ())),
        preferred_element_type=jnp.float32,
    )


def _yy_matmul(emb, wstk):
    blk = 400
    return pl.pallas_call(
        _yy_body,
        grid=(8, N // blk),
        in_specs=[
            pl.BlockSpec((blk, H), lambda k, n: (n, 0)),
            pl.BlockSpec((1, H, H), lambda k, n: (k, 0, 0)),
        ],
        out_specs=pl.BlockSpec((blk, H), lambda k, n: (k * (N // blk) + n, 0)),
        out_shape=jax.ShapeDtypeStruct((8 * N, H), jnp.float32),
    )(emb, wstk)


def _combine_body(p_ref, o_ref):
    o_ref[...] = p_ref[0] + p_ref[1]


def _combine(partials):
    blk = 400
    return pl.pallas_call(
        _combine_body,
        grid=(N // blk,),
        in_specs=[pl.BlockSpec((2, blk, H), lambda i: (0, i, 0))],
        out_specs=pl.BlockSpec((blk, H), lambda i: (i, 0)),
        out_shape=jax.ShapeDtypeStruct((N, H), jnp.float32),
    )(partials)


def _sc_scatter(yy, g3, d3):
    mesh = plsc.VectorSubcoreMesh(core_axis_name="c", subcore_axis_name="s")

    @functools.partial(
        pl.kernel,
        out_type=jax.ShapeDtypeStruct((NC, ACC_ROWS, H), jnp.float32),
        mesh=mesh,
        scratch_types=[
            pltpu.VMEM((IDX_CHUNK, BATCH), jnp.int32),
            pltpu.VMEM((IDX_CHUNK, BATCH), jnp.int32),
            pltpu.VMEM((BATCH, H), jnp.float32),
            pltpu.VMEM((BATCH, H), jnp.float32),
            pltpu.VMEM_SHARED((ACC_ROWS, H), jnp.float32),
            pltpu.SemaphoreType.DMA,
            pltpu.SemaphoreType.DMA,
        ],
    )
    def k(yy_hbm, g_hbm, d_hbm, out_hbm, g_v, d_v, rows_v, rows_w, acc, sem0, sem1):
        cid = lax.axis_index("c")
        sid = lax.axis_index("s")
        wid = cid * NS + sid

        zero = jnp.zeros((16,), jnp.float32)

        @pl.loop(0, BATCH)
        def _(r):
            @pl.loop(0, H // 16)
            def _(c):
                rows_v[r, pl.ds(c * 16, 16)] = zero

        @pl.loop(0, ROWS_PER_TILE // BATCH)
        def _(b):
            pltpu.sync_copy(
                rows_v, acc.at[pl.ds(sid * ROWS_PER_TILE + b * BATCH, BATCH)]
            )

        plsc.subcore_barrier()

        def gather(j, buf, sem):
            return pltpu.make_async_copy(yy_hbm.at[g_v.at[j]], buf, sem)

        @pl.loop(0, NB // IDX_CHUNK)
        def _(c):
            pltpu.sync_copy(g_hbm.at[wid, pl.ds(c * IDX_CHUNK, IDX_CHUNK)], g_v)
            pltpu.sync_copy(d_hbm.at[wid, pl.ds(c * IDX_CHUNK, IDX_CHUNK)], d_v)

            gather(0, rows_v, sem0).start()
            gather(1, rows_w, sem1).start()

            @pl.loop(0, IDX_CHUNK // 2 - 1)
            def _(b2):
                j0 = 2 * b2
                gather(j0, rows_v, sem0).wait()
                pltpu.sync_copy(rows_v, acc.at[d_v.at[j0]], add=True)
                gather(j0 + 2, rows_v, sem0).start()
                gather(j0 + 1, rows_w, sem1).wait()
                pltpu.sync_copy(rows_w, acc.at[d_v.at[j0 + 1]], add=True)
                gather(j0 + 3, rows_w, sem1).start()

            gather(IDX_CHUNK - 2, rows_v, sem0).wait()
            pltpu.sync_copy(rows_v, acc.at[d_v.at[IDX_CHUNK - 2]], add=True)
            gather(IDX_CHUNK - 1, rows_w, sem1).wait()
            pltpu.sync_copy(rows_w, acc.at[d_v.at[IDX_CHUNK - 1]], add=True)

        plsc.subcore_barrier()

        @pl.loop(0, ROWS_PER_TILE // BATCH)
        def _(b):
            base = sid * ROWS_PER_TILE + b * BATCH
            pltpu.sync_copy(
                acc.at[pl.ds(base, BATCH)], out_hbm.at[cid, pl.ds(base, BATCH)]
            )

    return k(yy, g3, d3)


@jax.jit
def kernel(state_embedding, typed_edges, W):
    et = typed_edges[0] - 1
    src = typed_edges[1]
    dst = typed_edges[2]

    g = jnp.concatenate([et * N + src, 4 * N + et * N + dst])
    d = jnp.concatenate([dst, dst])
    pad = PAIRS_PAD - PAIRS
    g = jnp.concatenate([g, jnp.zeros((pad,), jnp.int32)])
    d = jnp.concatenate([d, jnp.full((pad,), DUMP, jnp.int32)])
    g3 = g.reshape(NB, NW, BATCH).swapaxes(0, 1)
    d3 = d.reshape(NB, NW, BATCH).swapaxes(0, 1)

    wstk = jnp.concatenate([W[:, :, :H], W[:, :, H:]], axis=0)

    yy = _yy_matmul(state_embedding, wstk)
    partials = _sc_scatter(yy, g3, d3)
    return _combine(partials)


# speedup vs baseline: 10.9564x; 1.5896x over previous
import dataclasses
import functools

import jax
import jax.numpy as jnp
from jax import lax
from jax.experimental import pallas as pl
from jax.experimental.pallas import tpu as pltpu
from jax.experimental.pallas import tpu_sc as plsc

N = 10000
E = 320000
H = 128
T = 4

NC = 2
NS = 16
NW = NC * NS

BATCH = 128
NB = 80
IDX_CHUNK = 8
PAIRS_PAD = NW * NB * BATCH
ROWS_PER_TILE = 640
ACC_ROWS = NS * ROWS_PER_TILE
CNT_W = 16
DUMP = N

_MESH = plsc.VectorSubcoreMesh(core_axis_name="c", subcore_axis_name="s")
_CP = pltpu.CompilerParams()
if "needs_layout_passes" in pltpu.CompilerParams.__dataclass_fields__:
    _CP = dataclasses.replace(_CP, needs_layout_passes=False)


def _yy_body(emb_ref, w_ref, out_ref):
    out_ref[...] = lax.dot_general(
        emb_ref[...], w_ref[0],
        (((1,), (1,)), ((), ())),
        preferred_element_type=jnp.float32,
    )


def _yy_matmul(emb, wstk):
    blk = 400
    return pl.pallas_call(
        _yy_body,
        grid=(8, N // blk),
        in_specs=[
            pl.BlockSpec((blk, H), lambda k, n: (n, 0)),
            pl.BlockSpec((1, H, H), lambda k, n: (k, 0, 0)),
        ],
        out_specs=pl.BlockSpec((blk, H), lambda k, n: (k * (N // blk) + n, 0)),
        out_shape=jax.ShapeDtypeStruct((8 * N, H), jnp.float32),
    )(emb, wstk)


def _combine_body(p_ref, c_ref, yd_ref, o_ref):
    cnt = c_ref[0] + c_ref[1]
    acc = p_ref[0] + p_ref[1]
    for t in range(T):
        acc = acc + cnt[:, t:t + 1] * yd_ref[t]
    o_ref[...] = acc


def _combine(partials, cparts, ydst):
    blk = 400
    return pl.pallas_call(
        _combine_body,
        grid=(N // blk,),
        in_specs=[
            pl.BlockSpec((2, blk, H), lambda i: (0, i, 0)),
            pl.BlockSpec((2, blk, CNT_W), lambda i: (0, i, 0)),
            pl.BlockSpec((T, blk, H), lambda i: (0, i, 0)),
        ],
        out_specs=pl.BlockSpec((blk, H), lambda i: (i, 0)),
        out_shape=jax.ShapeDtypeStruct((N, H), jnp.float32),
    )(partials, cparts, ydst)


def _sc_rows(yy, g3, d3):
    @functools.partial(
        pl.kernel,
        compiler_params=_CP,
        out_type=jax.ShapeDtypeStruct((NC, ACC_ROWS, H), jnp.float32),
        mesh=_MESH,
        scratch_types=[
            pltpu.VMEM((IDX_CHUNK, BATCH), jnp.int32),
            pltpu.VMEM((IDX_CHUNK, BATCH), jnp.int32),
            pltpu.VMEM((BATCH, H), jnp.float32),
            pltpu.VMEM((BATCH, H), jnp.float32),
            pltpu.VMEM_SHARED((ACC_ROWS, H), jnp.float32),
            pltpu.SemaphoreType.DMA,
            pltpu.SemaphoreType.DMA,
        ],
    )
    def k(yy_hbm, g_hbm, d_hbm, out_hbm, g_v, d_v, rows_v, rows_w, acc,
          sem0, sem1):
        cid = lax.axis_index("c")
        sid = lax.axis_index("s")
        wid = cid * NS + sid

        zero = jnp.zeros((16,), jnp.float32)

        @pl.loop(0, BATCH)
        def _(r):
            @pl.loop(0, H // 16)
            def _(c):
                rows_v[r, pl.ds(c * 16, 16)] = zero

        @pl.loop(0, ROWS_PER_TILE // BATCH)
        def _(b):
            pltpu.sync_copy(
                rows_v, acc.at[pl.ds(sid * ROWS_PER_TILE + b * BATCH, BATCH)]
            )

        plsc.subcore_barrier()

        def gather(j, buf, sem):
            return pltpu.make_async_copy(yy_hbm.at[g_v.at[j]], buf, sem)

        @pl.loop(0, NB // IDX_CHUNK)
        def _(c):
            pltpu.sync_copy(g_hbm.at[wid, pl.ds(c * IDX_CHUNK, IDX_CHUNK)], g_v)
            pltpu.sync_copy(d_hbm.at[wid, pl.ds(c * IDX_CHUNK, IDX_CHUNK)], d_v)

            gather(0, rows_v, sem0).start()
            gather(1, rows_w, sem1).start()

            @pl.loop(0, IDX_CHUNK // 2 - 1)
            def _(b2):
                j0 = 2 * b2
                gather(j0, rows_v, sem0).wait()
                pltpu.sync_copy(rows_v, acc.at[d_v.at[j0]], add=True)
                gather(j0 + 2, rows_v, sem0).start()
                gather(j0 + 1, rows_w, sem1).wait()
                pltpu.sync_copy(rows_w, acc.at[d_v.at[j0 + 1]], add=True)
                gather(j0 + 3, rows_w, sem1).start()

            gather(IDX_CHUNK - 2, rows_v, sem0).wait()
            pltpu.sync_copy(rows_v, acc.at[d_v.at[IDX_CHUNK - 2]], add=True)
            gather(IDX_CHUNK - 1, rows_w, sem1).wait()
            pltpu.sync_copy(rows_w, acc.at[d_v.at[IDX_CHUNK - 1]], add=True)

        plsc.subcore_barrier()

        @pl.loop(0, ROWS_PER_TILE // BATCH)
        def _(b):
            base = sid * ROWS_PER_TILE + b * BATCH
            pltpu.sync_copy(
                acc.at[pl.ds(base, BATCH)], out_hbm.at[cid, pl.ds(base, BATCH)]
            )

    return k(yy, g3, d3)


CROWS = NS * NB
CRPT = CROWS // NS


def _sc_counts(c3, s3):
    @functools.partial(
        pl.kernel,
        compiler_params=_CP,
        out_type=jax.ShapeDtypeStruct((NC, CROWS, H), jnp.float32),
        mesh=_MESH,
        scratch_types=[
            pltpu.VMEM((IDX_CHUNK, BATCH), jnp.int32),
            pltpu.VMEM((IDX_CHUNK, BATCH), jnp.int32),
            pltpu.VMEM((BATCH, H), jnp.float32),
            pltpu.VMEM_SHARED((CROWS, H), jnp.float32),
        ],
    )
    def k(c_hbm, s_hbm, cnt_hbm, c_v, s_v, oh_v, cac):
        cid = lax.axis_index("c")
        sid = lax.axis_index("s")
        wid = cid * NS + sid

        zero = jnp.zeros((16,), jnp.float32)
        ones = jnp.ones((16,), jnp.float32)

        @pl.loop(0, BATCH)
        def _(r):
            @pl.loop(0, H // 16)
            def _(c):
                oh_v[r, pl.ds(c * 16, 16)] = zero

        pltpu.sync_copy(oh_v.at[pl.ds(0, CRPT)], cac.at[pl.ds(sid * CRPT, CRPT)])
        plsc.subcore_barrier()

        @pl.loop(0, NB // IDX_CHUNK)
        def _(c):
            pltpu.sync_copy(c_hbm.at[wid, pl.ds(c * IDX_CHUNK, IDX_CHUNK)], c_v)
            pltpu.sync_copy(s_hbm.at[wid, pl.ds(c * IDX_CHUNK, IDX_CHUNK)], s_v)

            @pl.loop(0, IDX_CHUNK)
            def _(j):
                for l in range(BATCH // 16):
                    rows = lax.iota(jnp.int32, 16) + (l * 16)
                    colv = c_v.at[j][pl.ds(l * 16, 16)]
                    plsc.addupdate_scatter(oh_v, [rows, colv], ones)
                pltpu.sync_copy(oh_v, cac.at[s_v.at[j]], add=True)
                for l in range(BATCH // 16):
                    rows = lax.iota(jnp.int32, 16) + (l * 16)
                    colv = c_v.at[j][pl.ds(l * 16, 16)]
                    plsc.addupdate_scatter(oh_v, [rows, colv], -ones)

        plsc.subcore_barrier()

        pltpu.sync_copy(cac.at[pl.ds(sid * CRPT, CRPT)],
                        cnt_hbm.at[cid, pl.ds(sid * CRPT, CRPT)])

    return k(c3, s3)


@jax.jit
def kernel(state_embedding, typed_edges, W):
    et = typed_edges[0] - 1
    src = typed_edges[1]
    dst = typed_edges[2]

    pad = PAIRS_PAD - E

    def shard(x, padval):
        x = jnp.concatenate([x, jnp.full((pad,), padval, jnp.int32)])
        return x.reshape(NB, NW, BATCH).swapaxes(0, 1)

    g3 = shard(et * N + src, 0)
    d3 = shard(dst, DUMP)
    c3 = shard(((dst & 7) << 4) + et, 0)
    s3 = shard(dst >> 3, DUMP >> 3)

    wstk = jnp.concatenate([W[:, :, :H], W[:, :, H:]], axis=0)

    yy = _yy_matmul(state_embedding, wstk)
    cparts = _sc_counts(c3, s3)
    partials = _sc_rows(yy, g3, d3)
    ydst = yy.reshape(8, N, H)[T:]
    cnt = cparts.reshape(2, CROWS * 8, CNT_W)
    return _combine(partials, cnt, ydst)


# speedup vs baseline: 10.9985x; 1.0038x over previous
import dataclasses
import functools

import jax
import jax.numpy as jnp
from jax import lax
from jax.experimental import pallas as pl
from jax.experimental.pallas import tpu as pltpu
from jax.experimental.pallas import tpu_sc as plsc

N = 10000
E = 320000
H = 128
T = 4

NC = 2
NS = 16
NW = NC * NS

BATCH = 128
NB = 80
IDX_CHUNK = 8
RB = 64
RNB = 160
RIDX = 16
NBUF = 4
PAIRS_PAD = NW * NB * BATCH
ROWS_PER_TILE = 640
ACC_ROWS = NS * ROWS_PER_TILE
CNT_W = 16
DUMP = N

_MESH = plsc.VectorSubcoreMesh(core_axis_name="c", subcore_axis_name="s")
_CP = pltpu.CompilerParams()
if "needs_layout_passes" in pltpu.CompilerParams.__dataclass_fields__:
    _CP = dataclasses.replace(_CP, needs_layout_passes=False)


def _yy_body(emb_ref, w_ref, out_ref):
    out_ref[...] = lax.dot_general(
        emb_ref[...], w_ref[0],
        (((1,), (1,)), ((), ())),
        preferred_element_type=jnp.float32,
    )


def _yy_matmul(emb, wstk):
    blk = 400
    return pl.pallas_call(
        _yy_body,
        grid=(8, N // blk),
        in_specs=[
            pl.BlockSpec((blk, H), lambda k, n: (n, 0)),
            pl.BlockSpec((1, H, H), lambda k, n: (k, 0, 0)),
        ],
        out_specs=pl.BlockSpec((blk, H), lambda k, n: (k * (N // blk) + n, 0)),
        out_shape=jax.ShapeDtypeStruct((8 * N, H), jnp.float32),
    )(emb, wstk)


def _combine_body(p_ref, c_ref, yd_ref, o_ref):
    cnt = c_ref[0] + c_ref[1]
    acc = p_ref[0] + p_ref[1]
    for t in range(T):
        acc = acc + cnt[:, t:t + 1] * yd_ref[t]
    o_ref[...] = acc


def _combine(partials, cparts, ydst):
    blk = 400
    return pl.pallas_call(
        _combine_body,
        grid=(N // blk,),
        in_specs=[
            pl.BlockSpec((2, blk, H), lambda i: (0, i, 0)),
            pl.BlockSpec((2, blk, CNT_W), lambda i: (0, i, 0)),
            pl.BlockSpec((T, blk, H), lambda i: (0, i, 0)),
        ],
        out_specs=pl.BlockSpec((blk, H), lambda i: (i, 0)),
        out_shape=jax.ShapeDtypeStruct((N, H), jnp.float32),
    )(partials, cparts, ydst)


def _sc_rows(yy, g3, d3):
    @functools.partial(
        pl.kernel,
        compiler_params=_CP,
        out_type=jax.ShapeDtypeStruct((NC, ACC_ROWS, H), jnp.float32),
        mesh=_MESH,
        scratch_types=[
            pltpu.VMEM((RIDX, RB), jnp.int32),
            pltpu.VMEM((RIDX, RB), jnp.int32),
        ] + [pltpu.VMEM((RB, H), jnp.float32) for _ in range(NBUF)] + [
            pltpu.VMEM_SHARED((ACC_ROWS, H), jnp.float32),
        ] + [pltpu.SemaphoreType.DMA for _ in range(2 * NBUF)],
    )
    def k(yy_hbm, g_hbm, d_hbm, out_hbm, g_v, d_v, *rest):
        bufs = rest[:NBUF]
        acc = rest[NBUF]
        gsem = rest[NBUF + 1:NBUF + 1 + NBUF]
        ssem = rest[NBUF + 1 + NBUF:]
        cid = lax.axis_index("c")
        sid = lax.axis_index("s")
        wid = cid * NS + sid

        zero = jnp.zeros((16,), jnp.float32)

        @pl.loop(0, RB)
        def _(r):
            @pl.loop(0, H // 16)
            def _(c):
                bufs[0][r, pl.ds(c * 16, 16)] = zero

        @pl.loop(0, ROWS_PER_TILE // RB)
        def _(b):
            pltpu.sync_copy(
                bufs[0], acc.at[pl.ds(sid * ROWS_PER_TILE + b * RB, RB)]
            )

        plsc.subcore_barrier()

        def gather(j, b):
            return pltpu.make_async_copy(yy_hbm.at[g_v.at[j]], bufs[b], gsem[b])

        def scat(j, b):
            return pltpu.make_async_copy(bufs[b], acc.at[d_v.at[j]], ssem[b])

        @pl.loop(0, RNB // RIDX)
        def _(c):
            pltpu.sync_copy(g_hbm.at[wid, pl.ds(c * RIDX, RIDX)], g_v)
            pltpu.sync_copy(d_hbm.at[wid, pl.ds(c * RIDX, RIDX)], d_v)

            for j in range(NBUF):
                gather(j, j).start()
            for j in range(RIDX):
                b = j % NBUF
                if j > 0:
                    pb = (j - 1) % NBUF
                    scat(j - 1, pb).wait()
                    if j + NBUF - 1 < RIDX:
                        gather(j + NBUF - 1, pb).start()
                gather(j, b).wait()
                scat(j, b).start(add=True)
            scat(RIDX - 1, (RIDX - 1) % NBUF).wait()

        plsc.subcore_barrier()

        @pl.loop(0, ROWS_PER_TILE // BATCH)
        def _(b):
            base = sid * ROWS_PER_TILE + b * BATCH
            pltpu.sync_copy(
                acc.at[pl.ds(base, BATCH)], out_hbm.at[cid, pl.ds(base, BATCH)]
            )

    return k(yy, g3, d3)


CROWS = NS * NB
CRPT = CROWS // NS


def _sc_counts(c3, s3):
    @functools.partial(
        pl.kernel,
        compiler_params=_CP,
        out_type=jax.ShapeDtypeStruct((NC, CROWS, H), jnp.float32),
        mesh=_MESH,
        scratch_types=[
            pltpu.VMEM((IDX_CHUNK, BATCH), jnp.int32),
            pltpu.VMEM((IDX_CHUNK, BATCH), jnp.int32),
            pltpu.VMEM((BATCH, H), jnp.float32),
            pltpu.VMEM_SHARED((CROWS, H), jnp.float32),
        ],
    )
    def k(c_hbm, s_hbm, cnt_hbm, c_v, s_v, oh_v, cac):
        cid = lax.axis_index("c")
        sid = lax.axis_index("s")
        wid = cid * NS + sid

        zero = jnp.zeros((16,), jnp.float32)
        ones = jnp.ones((16,), jnp.float32)

        @pl.loop(0, BATCH)
        def _(r):
            @pl.loop(0, H // 16)
            def _(c):
                oh_v[r, pl.ds(c * 16, 16)] = zero

        pltpu.sync_copy(oh_v.at[pl.ds(0, CRPT)], cac.at[pl.ds(sid * CRPT, CRPT)])
        plsc.subcore_barrier()

        @pl.loop(0, NB // IDX_CHUNK)
        def _(c):
            pltpu.sync_copy(c_hbm.at[wid, pl.ds(c * IDX_CHUNK, IDX_CHUNK)], c_v)
            pltpu.sync_copy(s_hbm.at[wid, pl.ds(c * IDX_CHUNK, IDX_CHUNK)], s_v)

            @pl.loop(0, IDX_CHUNK)
            def _(j):
                for l in range(BATCH // 16):
                    rows = lax.iota(jnp.int32, 16) + (l * 16)
                    colv = c_v.at[j][pl.ds(l * 16, 16)]
                    plsc.addupdate_scatter(oh_v, [rows, colv], ones)
                pltpu.sync_copy(oh_v, cac.at[s_v.at[j]], add=True)
                for l in range(BATCH // 16):
                    rows = lax.iota(jnp.int32, 16) + (l * 16)
                    colv = c_v.at[j][pl.ds(l * 16, 16)]
                    plsc.addupdate_scatter(oh_v, [rows, colv], -ones)

        plsc.subcore_barrier()

        pltpu.sync_copy(cac.at[pl.ds(sid * CRPT, CRPT)],
                        cnt_hbm.at[cid, pl.ds(sid * CRPT, CRPT)])

    return k(c3, s3)


@jax.jit
def kernel(state_embedding, typed_edges, W):
    et = typed_edges[0] - 1
    src = typed_edges[1]
    dst = typed_edges[2]

    def shard(x, padval, nb, batch):
        pad = NW * nb * batch - E
        x = jnp.concatenate([x, jnp.full((pad,), padval, jnp.int32)])
        return x.reshape(nb, NW, batch).swapaxes(0, 1)

    g3 = shard(et * N + src, 0, RNB, RB)
    d3 = shard(dst, DUMP, RNB, RB)
    c3 = shard(((dst & 7) << 4) + et, 0, NB, BATCH)
    s3 = shard(dst >> 3, DUMP >> 3, NB, BATCH)

    wstk = jnp.concatenate([W[:, :, :H], W[:, :, H:]], axis=0)

    yy = _yy_matmul(state_embedding, wstk)
    cparts = _sc_counts(c3, s3)
    partials = _sc_rows(yy, g3, d3)
    ydst = yy.reshape(8, N, H)[T:]
    cnt = cparts.reshape(2, CROWS * 8, CNT_W)
    return _combine(partials, cnt, ydst)
